# baseline (speedup 1.0000x reference)
import jax
import jax.numpy as jnp
from jax.experimental import pallas as pl
from jax.experimental.pallas import tpu as pltpu

_N_ATOMS = 32768
_N_MOL = 16
_NODE_DIM = 128
_HIDDEN_DIM = 64
_N_ELEM = 119
_TILE = 16384
_GRID = _N_ATOMS // _TILE


def _spatial_kernel(x_ref, posT_ref, batch_ref, atno_ref, mt_ref, W1_ref,
                    b1_ref, W2_ref, b2_ref, out_ref, accs_ref, accm_ref):
    i = pl.program_id(0)

    @pl.when(i == 0)
    def _init():
        accs_ref[...] = jnp.zeros_like(accs_ref)
        accm_ref[...] = jnp.zeros_like(accm_ref)

    x = x_ref[...]
    pT = posT_ref[...]
    b = batch_ref[...]
    a = atno_ref[...]

    el = jax.lax.broadcasted_iota(jnp.int32, (128, _TILE), 0)
    m = jnp.sum(jnp.where(el == a, mt_ref[...], 0.0), axis=0,
                keepdims=True)

    px = pT[0:1, :]
    py = pT[1:2, :]
    pz = pT[2:3, :]
    r2 = px * px + py * py + pz * pz

    seg = jnp.where(
        jax.lax.broadcasted_iota(jnp.int32, (_N_MOL, _TILE), 0) == b,
        1.0, 0.0)

    h = jnp.dot(x, W1_ref[...], preferred_element_type=jnp.float32)
    h = h + b1_ref[...]
    h = h * jax.nn.sigmoid(h)
    s = jnp.dot(h, W2_ref[...], preferred_element_type=jnp.float32)
    s = s + b2_ref[...]

    SF = jnp.concatenate([seg * r2, seg * px, seg * py, seg * pz, seg],
                         axis=0)
    accs_ref[...] += jax.lax.dot_general(
        SF, s, (((1,), (0,)), ((), ())),
        preferred_element_type=jnp.float32)

    MF = jnp.concatenate([seg * px, seg * py, seg * pz, seg],
                         axis=0)
    accm_ref[...] += jax.lax.dot_general(
        MF, m, (((1,), (1,)), ((), ())),
        preferred_element_type=jnp.float32)

    @pl.when(i == _GRID - 1)
    def _finalize():
        A = accs_ref[0:16, :]
        Bx = accs_ref[16:32, :]
        By = accs_ref[32:48, :]
        Bz = accs_ref[48:64, :]
        C = accs_ref[64:80, :]
        Mx = accm_ref[0:16, :]
        My = accm_ref[16:32, :]
        Mz = accm_ref[32:48, :]
        S = accm_ref[48:64, :]
        den = jnp.where(S > 0.0, S, 1.0)
        cx = Mx / den
        cy = My / den
        cz = Mz / den
        res = (A - 2.0 * (Bx * cx + By * cy + Bz * cz)
               + C * (cx * cx + cy * cy + cz * cz))
        out_ref[...] = res


def kernel(x_scalar, x_spherical, pos, batch, at_no, masses_table, W1, b1,
           W2, b2):
    del x_spherical
    posT = pos.T
    batch2 = batch.astype(jnp.int32).reshape(1, _N_ATOMS)
    atno2 = at_no.astype(jnp.int32).reshape(1, _N_ATOMS)
    mt = jnp.zeros((128, 1), jnp.float32).at[:_N_ELEM, 0].set(masses_table)
    b1r = b1.reshape(1, _HIDDEN_DIM)
    b2r = b2.reshape(1, 1)

    out = pl.pallas_call(
        _spatial_kernel,
        grid=(_GRID,),
        in_specs=[
            pl.BlockSpec((_TILE, _NODE_DIM), lambda i: (i, 0)),
            pl.BlockSpec((3, _TILE), lambda i: (0, i)),
            pl.BlockSpec((1, _TILE), lambda i: (0, i)),
            pl.BlockSpec((1, _TILE), lambda i: (0, i)),
            pl.BlockSpec((128, 1), lambda i: (0, 0)),
            pl.BlockSpec((_NODE_DIM, _HIDDEN_DIM), lambda i: (0, 0)),
            pl.BlockSpec((1, _HIDDEN_DIM), lambda i: (0, 0)),
            pl.BlockSpec((_HIDDEN_DIM, 1), lambda i: (0, 0)),
            pl.BlockSpec((1, 1), lambda i: (0, 0)),
        ],
        out_specs=pl.BlockSpec((_N_MOL, 1), lambda i: (0, 0)),
        out_shape=jax.ShapeDtypeStruct((_N_MOL, 1), jnp.float32),
        scratch_shapes=[pltpu.VMEM((80, 1), jnp.float32),
                        pltpu.VMEM((64, 1), jnp.float32)],
        compiler_params=pltpu.CompilerParams(
            dimension_semantics=("arbitrary",)),
    )(x_scalar, posT, batch2, atno2, mt, W1, b1r, W2, b2r)
    return out


# speedup vs baseline: 1.0294x; 1.0294x over previous
import jax
import jax.numpy as jnp
from jax.experimental import pallas as pl
from jax.experimental.pallas import tpu as pltpu

_N_ATOMS = 32768
_N_MOL = 16
_NODE_DIM = 128
_HIDDEN_DIM = 64
_N_ELEM = 119
_TILE = 8192
_GRID = _N_ATOMS // _TILE


def _spatial_kernel(x_ref, posT_ref, batch_ref, atno_ref, mt_ref, W1_ref,
                    b1_ref, W2_ref, b2_ref, out_ref, accs_ref, accm_ref):
    i = pl.program_id(0)

    @pl.when(i == 0)
    def _init():
        accs_ref[...] = jnp.zeros_like(accs_ref)
        accm_ref[...] = jnp.zeros_like(accm_ref)

    x = x_ref[...]
    pT = posT_ref[...]
    b = batch_ref[...]
    a = atno_ref[...]

    el = jax.lax.broadcasted_iota(jnp.int32, (128, _TILE), 0)
    m = jnp.sum(jnp.where(el == a, mt_ref[...], 0.0), axis=0,
                keepdims=True)

    px = pT[0:1, :]
    py = pT[1:2, :]
    pz = pT[2:3, :]
    r2 = px * px + py * py + pz * pz

    seg = jnp.where(
        jax.lax.broadcasted_iota(jnp.int32, (_N_MOL, _TILE), 0) == b,
        1.0, 0.0)

    h = jnp.dot(x, W1_ref[...], preferred_element_type=jnp.float32)
    h = h + b1_ref[...]
    h = h * jax.nn.sigmoid(h)
    s = jnp.dot(h, W2_ref[...], preferred_element_type=jnp.float32)
    s = s + b2_ref[...]

    SF = jnp.concatenate([seg * r2, seg * px, seg * py, seg * pz, seg],
                         axis=0)
    accs_ref[...] += jax.lax.dot_general(
        SF, s, (((1,), (0,)), ((), ())),
        preferred_element_type=jnp.float32)

    MF = jnp.concatenate([seg * px, seg * py, seg * pz, seg],
                         axis=0)
    accm_ref[...] += jax.lax.dot_general(
        MF, m, (((1,), (1,)), ((), ())),
        preferred_element_type=jnp.float32)

    @pl.when(i == _GRID - 1)
    def _finalize():
        A = accs_ref[0:16, :]
        Bx = accs_ref[16:32, :]
        By = accs_ref[32:48, :]
        Bz = accs_ref[48:64, :]
        C = accs_ref[64:80, :]
        Mx = accm_ref[0:16, :]
        My = accm_ref[16:32, :]
        Mz = accm_ref[32:48, :]
        S = accm_ref[48:64, :]
        den = jnp.where(S > 0.0, S, 1.0)
        cx = Mx / den
        cy = My / den
        cz = Mz / den
        res = (A - 2.0 * (Bx * cx + By * cy + Bz * cz)
               + C * (cx * cx + cy * cy + cz * cz))
        out_ref[...] = res


def kernel(x_scalar, x_spherical, pos, batch, at_no, masses_table, W1, b1,
           W2, b2):
    del x_spherical
    posT = pos.T
    batch2 = batch.astype(jnp.int32).reshape(1, _N_ATOMS)
    atno2 = at_no.astype(jnp.int32).reshape(1, _N_ATOMS)
    mt = jnp.zeros((128, 1), jnp.float32).at[:_N_ELEM, 0].set(masses_table)
    b1r = b1.reshape(1, _HIDDEN_DIM)
    b2r = b2.reshape(1, 1)

    out = pl.pallas_call(
        _spatial_kernel,
        grid=(_GRID,),
        in_specs=[
            pl.BlockSpec((_TILE, _NODE_DIM), lambda i: (i, 0)),
            pl.BlockSpec((3, _TILE), lambda i: (0, i)),
            pl.BlockSpec((1, _TILE), lambda i: (0, i)),
            pl.BlockSpec((1, _TILE), lambda i: (0, i)),
            pl.BlockSpec((128, 1), lambda i: (0, 0)),
            pl.BlockSpec((_NODE_DIM, _HIDDEN_DIM), lambda i: (0, 0)),
            pl.BlockSpec((1, _HIDDEN_DIM), lambda i: (0, 0)),
            pl.BlockSpec((_HIDDEN_DIM, 1), lambda i: (0, 0)),
            pl.BlockSpec((1, 1), lambda i: (0, 0)),
        ],
        out_specs=pl.BlockSpec((_N_MOL, 1), lambda i: (0, 0)),
        out_shape=jax.ShapeDtypeStruct((_N_MOL, 1), jnp.float32),
        scratch_shapes=[pltpu.VMEM((80, 1), jnp.float32),
                        pltpu.VMEM((64, 1), jnp.float32)],
        compiler_params=pltpu.CompilerParams(
            dimension_semantics=("arbitrary",)),
    )(x_scalar, posT, batch2, atno2, mt, W1, b1r, W2, b2r)
    return out
